# Initial kernel scaffold; baseline (speedup 1.0000x reference)
import functools

import jax
import jax.numpy as jnp
from jax import lax
from jax.experimental import pallas as pl
from jax.experimental.pallas import tpu as pltpu
from jax.experimental.pallas import tpu_sc as plsc

N = 50000
C = 128
KV = 27
P = 8192
TOT = KV * P
NC = 2
NS = 16
NW = NC * NS

G_ROWS = TOT // NW
G_CH = 128
G_NCH = G_ROWS // G_CH

TILE = 12500
S_ROWS = TOT // NS
S_IDXR = S_ROWS // 128
ACC_ROWS = 12544


def _gather_sc(feats, in_idx2d):
    mesh = plsc.VectorSubcoreMesh(core_axis_name="c", subcore_axis_name="s")

    @functools.partial(
        pl.kernel,
        out_type=jax.ShapeDtypeStruct((TOT, C), jnp.float32),
        mesh=mesh,
        scratch_types=[
            pltpu.VMEM((G_NCH, G_CH), jnp.int32),
            pltpu.VMEM((G_CH, C), jnp.float32),
            pltpu.SemaphoreType.DMA,
        ],
    )
    def k(feats_hbm, idx_hbm, out_hbm, idx_v, rows_v, sem):
        wid = lax.axis_index("s") * NC + lax.axis_index("c")
        rbase = wid * G_NCH
        pltpu.sync_copy(idx_hbm.at[pl.ds(rbase, G_NCH)], idx_v)

        @pl.loop(0, G_NCH)
        def _(j):
            pltpu.async_copy(feats_hbm.at[idx_v.at[j]], rows_v, sem).wait()
            pltpu.sync_copy(rows_v, out_hbm.at[pl.ds((rbase + j) * G_CH, G_CH)])

    return k(feats, in_idx2d)


def _gemm_tc(gathered, w):
    BP = 512

    def body(x_ref, w_ref, o_ref):
        o_ref[...] = jnp.dot(
            x_ref[0], w_ref[0], preferred_element_type=jnp.float32
        )[None]

    return pl.pallas_call(
        body,
        grid=(KV, P // BP),
        in_specs=[
            pl.BlockSpec((1, BP, C), lambda k, p: (k, p, 0)),
            pl.BlockSpec((1, C, C), lambda k, p: (k, 0, 0)),
        ],
        out_specs=pl.BlockSpec((1, BP, C), lambda k, p: (k, p, 0)),
        out_shape=jax.ShapeDtypeStruct((KV, P, C), jnp.float32),
    )(gathered, w)


def _scatter_sc(contrib, out_idx2d):
    mesh = plsc.VectorSubcoreMesh(core_axis_name="c", subcore_axis_name="s")

    @functools.partial(
        pl.kernel,
        out_type=jax.ShapeDtypeStruct((N, C), jnp.float32),
        mesh=mesh,
        scratch_types=[
            pltpu.VMEM((S_IDXR, 128), jnp.int32),
            pltpu.VMEM((S_IDXR, 128), jnp.int32),
            pltpu.VMEM((S_IDXR, 128), jnp.int32),
            pltpu.VMEM((128, C), jnp.float32),
            pltpu.VMEM((112, C), jnp.float32),
            pltpu.VMEM_SHARED((ACC_ROWS, C), jnp.float32),
            pltpu.SemaphoreType.DMA,
        ],
    )
    def k(contrib_hbm, idx_hbm, out_hbm, idxraw, loc, pos, rows, zbuf, acc, sem):
        cid = lax.axis_index("c")
        sid = lax.axis_index("s")
        pltpu.sync_copy(idx_hbm.at[pl.ds(sid * S_IDXR, S_IDXR)], idxraw)

        zero16f = jnp.zeros((16,), jnp.float32)
        zero16i = jnp.zeros((16,), jnp.int32)

        @pl.loop(0, 112)
        def _(r):
            @pl.loop(0, C, step=16)
            def _(cc):
                zbuf[r, pl.ds(cc, 16)] = zero16f

        lane = lax.iota(jnp.int32, 16)

        for t_local in range(2):
            base = (2 * cid + t_local) * TILE

            @pl.loop(0, 112)
            def _(m):
                @pl.when(lax.rem(m, NS) == sid)
                def _():
                    pltpu.sync_copy(zbuf, acc.at[pl.ds(m * 112, 112)])

            @pl.loop(0, S_IDXR)
            def _(r):
                @pl.loop(0, 128, step=16)
                def _(cc):
                    loc[r, pl.ds(cc, 16)] = zero16i
                    pos[r, pl.ds(cc, 16)] = zero16i

            def scan_row(r, cnt):
                for g in range(8):
                    col = g * 16
                    v = idxraw[r, pl.ds(col, 16)]
                    localv = v - base
                    maskv = (localv >= 0) & (localv < TILE)
                    mi = maskv.astype(jnp.int32)
                    pc = plsc.cumsum(mi)
                    q = cnt + pc - 1
                    row_i = lax.shift_right_logical(q, 7)
                    col_i = lax.bitwise_and(q, 127)
                    plsc.store_scatter(loc, [row_i, col_i], localv + 1,
                                       mask=maskv)
                    pv = (sid * S_ROWS + r * 128 + col) + lane
                    plsc.store_scatter(pos, [row_i, col_i], pv, mask=maskv)
                    cnt = cnt + jnp.sum(mi)
                return cnt

            cnt = lax.fori_loop(0, S_IDXR, scan_row, jnp.int32(0))
            nch = lax.shift_right_logical(cnt + 127, 7)

            plsc.subcore_barrier()

            def chunk_body(j, carry):
                pltpu.async_copy(contrib_hbm.at[pos.at[j]], rows, sem).wait()
                pltpu.sync_copy(rows, acc.at[loc.at[j]], add=True)
                return carry

            lax.fori_loop(0, nch, chunk_body, jnp.int32(0))

            plsc.subcore_barrier()

            @pl.loop(0, 100)
            def _(m):
                @pl.when(lax.rem(m, NS) == sid)
                def _():
                    pltpu.sync_copy(acc.at[pl.ds(1 + m * 125, 125)],
                                    out_hbm.at[pl.ds(base + m * 125, 125)])

            plsc.subcore_barrier()

    return k(contrib, out_idx2d)


def kernel(coords, feats, maps, mappat, kernel):
    w = kernel
    in_idx2d = maps[:, :, 0].reshape(TOT // 128, 128)
    out_idx2d = maps[:, :, 1].reshape(TOT // 128, 128)
    gathered = _gather_sc(feats, in_idx2d)
    contrib = _gemm_tc(gathered.reshape(KV, P, C), w)
    return _scatter_sc(contrib.reshape(TOT, C), out_idx2d)


# speedup vs baseline: 2.0240x; 2.0240x over previous
import dataclasses
import functools

import jax
import jax.numpy as jnp
from jax import lax
from jax.experimental import pallas as pl
from jax.experimental.pallas import tpu as pltpu
from jax.experimental.pallas import tpu_sc as plsc

N = 50000
C = 128
KV = 27
P = 8192
TOT = KV * P
NC = 2
NS = 16
NW = NC * NS

G_ROWS = TOT // NW
G_CH = 128
G_NCH = G_ROWS // G_CH

TILE = 12512
S_ROWS = TOT // NS
SEG = 1728
SEG_G = SEG // 16
CAP = 5248
CAP_CH = CAP // 128
DUMP = TILE
ACC_ROWS = 12544


def _gather_sc(feats, in_idx):
    mesh = plsc.VectorSubcoreMesh(core_axis_name="c", subcore_axis_name="s")

    @functools.partial(
        pl.kernel,
        out_type=jax.ShapeDtypeStruct((TOT, C), jnp.float32),
        mesh=mesh,
        scratch_types=[
            pltpu.VMEM((G_ROWS,), jnp.int32),
            pltpu.VMEM((G_CH, C), jnp.float32),
            pltpu.SemaphoreType.DMA,
        ],
    )
    def k(feats_hbm, idx_hbm, out_hbm, idx_v, rows_v, sem):
        wid = lax.axis_index("s") * NC + lax.axis_index("c")
        base = wid * G_ROWS
        pltpu.sync_copy(idx_hbm.at[pl.ds(base, G_ROWS)], idx_v)

        @pl.loop(0, G_NCH)
        def _(j):
            pltpu.async_copy(
                feats_hbm.at[idx_v.at[pl.ds(j * G_CH, G_CH)]], rows_v, sem
            ).wait()
            pltpu.sync_copy(rows_v, out_hbm.at[pl.ds(base + j * G_CH, G_CH)])

    return k(feats, in_idx)


def _gemm_tc(gathered, w):
    BP = 512

    def body(x_ref, w_ref, o_ref):
        o_ref[...] = jnp.dot(
            x_ref[0], w_ref[0], preferred_element_type=jnp.float32
        )[None]

    return pl.pallas_call(
        body,
        grid=(KV, P // BP),
        in_specs=[
            pl.BlockSpec((1, BP, C), lambda k, p: (k, p, 0)),
            pl.BlockSpec((1, C, C), lambda k, p: (k, 0, 0)),
        ],
        out_specs=pl.BlockSpec((1, BP, C), lambda k, p: (k, p, 0)),
        out_shape=jax.ShapeDtypeStruct((KV, P, C), jnp.float32),
    )(gathered, w)


def _sc_compiler_params():
    cp = pltpu.CompilerParams()
    if "needs_layout_passes" in pltpu.CompilerParams.__dataclass_fields__:
        cp = dataclasses.replace(cp, needs_layout_passes=False)
    return cp


def _scatter_sc(contrib, out_idx):
    mesh = plsc.VectorSubcoreMesh(core_axis_name="c", subcore_axis_name="s")

    @functools.partial(
        pl.kernel,
        out_type=jax.ShapeDtypeStruct((N, C), jnp.float32),
        mesh=mesh,
        compiler_params=_sc_compiler_params(),
        scratch_types=[
            pltpu.VMEM((SEG,), jnp.int32),
            pltpu.VMEM((CAP_CH, 128), jnp.int32),
            pltpu.VMEM((CAP_CH, 128), jnp.int32),
            pltpu.VMEM((128, C), jnp.float32),
            pltpu.VMEM_SHARED((ACC_ROWS, C), jnp.float32),
            pltpu.SemaphoreType.DMA,
        ],
    )
    def k(contrib_hbm, idx_hbm, out_hbm, idxseg, loc, pos, rows, acc, sem):
        cid = lax.axis_index("c")
        sid = lax.axis_index("s")

        zero16f = jnp.zeros((16,), jnp.float32)
        zero16i = jnp.zeros((16,), jnp.int32)
        dump16 = jnp.full((16,), DUMP, jnp.int32)
        lane = lax.iota(jnp.int32, 16)

        def process(cnt):
            top = lax.bitwise_and(cnt + 127, -128)
            for gi in range(8):
                q = cnt + gi * 16 + lane
                maskp = q < top
                row_i = lax.shift_right_logical(q, 7)
                col_i = lax.bitwise_and(q, 127)
                plsc.store_scatter(loc, [row_i, col_i], dump16, mask=maskp)
                plsc.store_scatter(pos, [row_i, col_i], zero16i, mask=maskp)

            def chunk_body(j, carry):
                pltpu.async_copy(contrib_hbm.at[pos.at[j]], rows, sem).wait()
                pltpu.sync_copy(rows, acc.at[loc.at[j]], add=True)
                return carry

            lax.fori_loop(0, lax.shift_right_logical(top, 7), chunk_body,
                          jnp.int32(0))
            return jnp.int32(0)

        for t_local in range(2):
            base = (2 * cid + t_local) * TILE
            rows_t = jnp.minimum(TILE, N - base)

            @pl.loop(0, 128)
            def _(r):
                @pl.loop(0, C, step=16)
                def _(cc):
                    rows[r, pl.ds(cc, 16)] = zero16f

            @pl.loop(0, ACC_ROWS // 128)
            def _(m):
                @pl.when(lax.rem(m, NS) == sid)
                def _():
                    pltpu.sync_copy(rows, acc.at[pl.ds(m * 128, 128)])

            plsc.subcore_barrier()

            cnt = jnp.int32(0)
            for g in range(8):
                cnt = lax.cond(cnt + SEG > CAP, process,
                               lambda c: c, cnt)
                pltpu.sync_copy(
                    idx_hbm.at[pl.ds(sid * S_ROWS + g * SEG, SEG)], idxseg)

                def scan_group(i, cnt):
                    col = i * 16
                    v = idxseg[pl.ds(col, 16)]
                    localv = v - base
                    maskv = (localv >= 0) & (localv < rows_t)
                    mi = maskv.astype(jnp.int32)
                    pc = plsc.cumsum(mi)
                    q = cnt + pc - 1
                    row_i = lax.shift_right_logical(q, 7)
                    col_i = lax.bitwise_and(q, 127)
                    plsc.store_scatter(loc, [row_i, col_i], localv,
                                       mask=maskv)
                    pv = (sid * S_ROWS + g * SEG + col) + lane
                    plsc.store_scatter(pos, [row_i, col_i], pv, mask=maskv)
                    return cnt + jnp.sum(mi)

                cnt = lax.fori_loop(0, SEG_G, scan_group, cnt)

            cnt = process(cnt)

            plsc.subcore_barrier()

            @pl.loop(0, 112)
            def _(m):
                @pl.when((lax.rem(m, NS) == sid) & ((m + 1) * 112 <= rows_t))
                def _():
                    pltpu.sync_copy(acc.at[pl.ds(m * 112, 112)],
                                    out_hbm.at[pl.ds(base + m * 112, 112)])

            for mt in range(5):
                @pl.when((sid == mt) & (12432 + (mt + 1) * 16 <= rows_t))
                def _():
                    pltpu.sync_copy(
                        acc.at[pl.ds(12432 + mt * 16, 16)],
                        out_hbm.at[pl.ds(base + 12432 + mt * 16, 16)])

            plsc.subcore_barrier()

    return k(contrib, out_idx)


def kernel(coords, feats, maps, mappat, kernel):
    w = kernel
    in_idx = maps[:, :, 0].reshape(TOT)
    out_idx = maps[:, :, 1].reshape(TOT)
    gathered = _gather_sc(feats, in_idx)
    contrib = _gemm_tc(gathered.reshape(KV, P, C), w)
    return _scatter_sc(contrib.reshape(TOT, C), out_idx)


# speedup vs baseline: 2.7621x; 1.3647x over previous
import dataclasses
import functools

import jax
import jax.numpy as jnp
from jax import lax
from jax.experimental import pallas as pl
from jax.experimental.pallas import tpu as pltpu
from jax.experimental.pallas import tpu_sc as plsc

N = 50000
C = 128
KV = 27
P = 8192
TOT = KV * P
NC = 2
NS = 16
NW = NC * NS

G_ROWS = TOT // NW
G_CH = 128
G_NCH = G_ROWS // G_CH

TILE = 12512
S_ROWS = TOT // NS
SEG = 1728
SEG_G = SEG // 16
CAP = 5248
CAP_CH = CAP // 128
DUMP = TILE
ACC_ROWS = 12544


def _gather_sc(feats, in_idx):
    mesh = plsc.VectorSubcoreMesh(core_axis_name="c", subcore_axis_name="s")

    @functools.partial(
        pl.kernel,
        out_type=jax.ShapeDtypeStruct((TOT, C), jnp.float32),
        mesh=mesh,
        scratch_types=[
            pltpu.VMEM((G_ROWS,), jnp.int32),
            pltpu.VMEM((G_CH, C), jnp.float32),
            pltpu.SemaphoreType.DMA,
        ],
    )
    def k(feats_hbm, idx_hbm, out_hbm, idx_v, rows_v, sem):
        wid = lax.axis_index("s") * NC + lax.axis_index("c")
        base = wid * G_ROWS
        pltpu.sync_copy(idx_hbm.at[pl.ds(base, G_ROWS)], idx_v)

        @pl.loop(0, G_NCH)
        def _(j):
            pltpu.async_copy(
                feats_hbm.at[idx_v.at[pl.ds(j * G_CH, G_CH)]], rows_v, sem
            ).wait()
            pltpu.sync_copy(rows_v, out_hbm.at[pl.ds(base + j * G_CH, G_CH)])

    return k(feats, in_idx)


def _gemm_tc(gathered, w):
    BP = 4096

    def body(x_ref, w_ref, o_ref):
        x = x_ref[0].astype(jnp.bfloat16)
        wb = w_ref[0].astype(jnp.bfloat16)
        o_ref[...] = jnp.dot(x, wb, preferred_element_type=jnp.float32)[None]

    return pl.pallas_call(
        body,
        grid=(KV, P // BP),
        in_specs=[
            pl.BlockSpec((1, BP, C), lambda k, p: (k, p, 0)),
            pl.BlockSpec((1, C, C), lambda k, p: (k, 0, 0)),
        ],
        out_specs=pl.BlockSpec((1, BP, C), lambda k, p: (k, p, 0)),
        out_shape=jax.ShapeDtypeStruct((KV, P, C), jnp.float32),
        compiler_params=pltpu.CompilerParams(
            dimension_semantics=("parallel", "arbitrary"),
        ),
    )(gathered, w)


def _sc_compiler_params():
    cp = pltpu.CompilerParams()
    if "needs_layout_passes" in pltpu.CompilerParams.__dataclass_fields__:
        cp = dataclasses.replace(cp, needs_layout_passes=False)
    return cp


def _scatter_sc(contrib, out_idx):
    mesh = plsc.VectorSubcoreMesh(core_axis_name="c", subcore_axis_name="s")

    @functools.partial(
        pl.kernel,
        out_type=jax.ShapeDtypeStruct((N, C), jnp.float32),
        mesh=mesh,
        compiler_params=_sc_compiler_params(),
        scratch_types=[
            pltpu.VMEM((SEG,), jnp.int32),
            pltpu.VMEM((CAP_CH, 128), jnp.int32),
            pltpu.VMEM((CAP_CH, 128), jnp.int32),
            pltpu.VMEM((128, C), jnp.float32),
            pltpu.VMEM_SHARED((ACC_ROWS, C), jnp.float32),
            pltpu.SemaphoreType.DMA,
        ],
    )
    def k(contrib_hbm, idx_hbm, out_hbm, idxseg, loc, pos, rows, acc, sem):
        cid = lax.axis_index("c")
        sid = lax.axis_index("s")

        zero16f = jnp.zeros((16,), jnp.float32)
        zero16i = jnp.zeros((16,), jnp.int32)
        dump16 = jnp.full((16,), DUMP, jnp.int32)
        lane = lax.iota(jnp.int32, 16)

        def process(cnt):
            top = lax.bitwise_and(cnt + 127, -128)
            for gi in range(8):
                q = cnt + gi * 16 + lane
                maskp = q < top
                row_i = lax.shift_right_logical(q, 7)
                col_i = lax.bitwise_and(q, 127)
                plsc.store_scatter(loc, [row_i, col_i], dump16, mask=maskp)
                plsc.store_scatter(pos, [row_i, col_i], zero16i, mask=maskp)

            def chunk_body(j, carry):
                pltpu.async_copy(contrib_hbm.at[pos.at[j]], rows, sem).wait()
                pltpu.sync_copy(rows, acc.at[loc.at[j]], add=True)
                return carry

            lax.fori_loop(0, lax.shift_right_logical(top, 7), chunk_body,
                          jnp.int32(0))
            return jnp.int32(0)

        for t_local in range(2):
            base = (2 * cid + t_local) * TILE
            rows_t = jnp.minimum(TILE, N - base)

            @pl.loop(0, 128)
            def _(r):
                @pl.loop(0, C, step=16)
                def _(cc):
                    rows[r, pl.ds(cc, 16)] = zero16f

            @pl.loop(0, ACC_ROWS // 128)
            def _(m):
                @pl.when(lax.rem(m, NS) == sid)
                def _():
                    pltpu.sync_copy(rows, acc.at[pl.ds(m * 128, 128)])

            plsc.subcore_barrier()

            cnt = jnp.int32(0)
            for g in range(8):
                cnt = lax.cond(cnt + SEG > CAP, process,
                               lambda c: c, cnt)
                pltpu.sync_copy(
                    idx_hbm.at[pl.ds(sid * S_ROWS + g * SEG, SEG)], idxseg)

                def scan_group(i, cnt):
                    col = i * 16
                    v = idxseg[pl.ds(col, 16)]
                    localv = v - base
                    maskv = (localv >= 0) & (localv < rows_t)
                    mi = maskv.astype(jnp.int32)
                    pc = plsc.cumsum(mi)
                    q = cnt + pc - 1
                    row_i = lax.shift_right_logical(q, 7)
                    col_i = lax.bitwise_and(q, 127)
                    plsc.store_scatter(loc, [row_i, col_i], localv,
                                       mask=maskv)
                    pv = (sid * S_ROWS + g * SEG + col) + lane
                    plsc.store_scatter(pos, [row_i, col_i], pv, mask=maskv)
                    return cnt + jnp.sum(mi)

                cnt = lax.fori_loop(0, SEG_G, scan_group, cnt)

            cnt = process(cnt)

            plsc.subcore_barrier()

            @pl.loop(0, 112)
            def _(m):
                @pl.when((lax.rem(m, NS) == sid) & ((m + 1) * 112 <= rows_t))
                def _():
                    pltpu.sync_copy(acc.at[pl.ds(m * 112, 112)],
                                    out_hbm.at[pl.ds(base + m * 112, 112)])

            for mt in range(5):
                @pl.when((sid == mt) & (12432 + (mt + 1) * 16 <= rows_t))
                def _():
                    pltpu.sync_copy(
                        acc.at[pl.ds(12432 + mt * 16, 16)],
                        out_hbm.at[pl.ds(base + 12432 + mt * 16, 16)])

            plsc.subcore_barrier()

    return k(contrib, out_idx)


def kernel(coords, feats, maps, mappat, kernel):
    w = kernel
    in_idx = maps[:, :, 0].reshape(TOT)
    out_idx = maps[:, :, 1].reshape(TOT)
    gathered = _gather_sc(feats, in_idx)
    contrib = _gemm_tc(gathered.reshape(KV, P, C), w)
    return _scatter_sc(contrib.reshape(TOT, C), out_idx)
